# baseline (speedup 1.0000x reference)
import math

import jax
import jax.numpy as jnp
from jax import lax
from jax.experimental import pallas as pl
from jax.experimental.pallas import tpu as pltpu
from jax.experimental.pallas import tpu_sc as plsc

D_MODEL = 64
SCALE = math.sqrt(D_MODEL)

NC = 2
NS = 16
NW = NC * NS
LANES = 16

GATHER = 128
NBUF = 2


def _build(B0, B1):
    assert B0 == GATHER * NW
    n_steps = B1
    mesh = plsc.VectorSubcoreMesh(
        core_axis_name="c", subcore_axis_name="s",
        num_cores=NC, num_subcores=NS)

    def body(xt_hbm, table_hbm, out_hbm, idx0, idx1, rows0, rows1, blk0, blk1,
             sem_g0, sem_g1, sem_w0, sem_w1):
        idx = (idx0, idx1)
        rows = (rows0, rows1)
        blk = (blk0, blk1)
        sem_g = (sem_g0, sem_g1)
        sem_w = (sem_w0, sem_w1)
        wid = lax.axis_index("s") * NC + lax.axis_index("c")
        col0 = wid * GATHER

        ilane = lax.iota(jnp.int32, LANES)

        def fire_gather(b1, b):
            pltpu.sync_copy(xt_hbm.at[b1, pl.ds(col0, GATHER)], idx[b])
            pltpu.async_copy(table_hbm.at[idx[b]], rows[b], sem_g[b])

        def wait_gather(b):
            pltpu.make_async_copy(
                table_hbm.at[idx[b]], rows[b], sem_g[b]).wait()

        def fire_wb(b1, b):
            for di in range(D_MODEL // 8):
                pltpu.async_copy(blk[b].at[di], out_hbm.at[b1, di, wid],
                                 sem_w[b])

        def wait_wb(b):
            for di in range(D_MODEL // 8):
                pltpu.make_async_copy(blk[b].at[di], out_hbm.at[0, di, 0],
                                      sem_w[b]).wait()

        def shuffle(b):
            src = rows[b]
            dst = blk[b]

            @plsc.parallel_loop(0, GATHER // LANES)
            def _(g):
                rvec = ilane + g * LANES
                for d in range(D_MODEL):
                    cvec = jnp.full((LANES,), d, jnp.int32)
                    v = plsc.load_gather(src, [rvec, cvec])
                    dst[d // 8, d % 8, pl.ds(g * LANES, LANES)] = v * SCALE

        for b in range(NBUF):
            fire_gather(b, b)

        def pair_body(b1_0, carry):
            for b in range(NBUF):
                b1 = b1_0 + b
                wait_gather(b)

                @pl.when(b1 >= NBUF)
                def _():
                    wait_wb(b)

                shuffle(b)
                fire_wb(b1, b)

                @pl.when(b1 + NBUF < n_steps)
                def _():
                    fire_gather(b1 + NBUF, b)
            return carry

        lax.fori_loop(0, n_steps // NBUF,
                      lambda i, c: pair_body(i * NBUF, c), 0)
        for b in range(NBUF):
            wait_wb(b)

    kern = pl.kernel(
        body,
        out_type=jax.ShapeDtypeStruct((B1, D_MODEL // 8, B0 // GATHER, 8, 128),
                                      jnp.float32),
        mesh=mesh,
        scratch_types=[
            pltpu.VMEM((GATHER,), jnp.int32),
            pltpu.VMEM((GATHER,), jnp.int32),
            pltpu.VMEM((GATHER, D_MODEL), jnp.float32),
            pltpu.VMEM((GATHER, D_MODEL), jnp.float32),
            pltpu.VMEM((D_MODEL // 8, 8, 128), jnp.float32),
            pltpu.VMEM((D_MODEL // 8, 8, 128), jnp.float32),
            pltpu.SemaphoreType.DMA,
            pltpu.SemaphoreType.DMA,
            pltpu.SemaphoreType.DMA,
            pltpu.SemaphoreType.DMA,
        ],
        compiler_params=pltpu.CompilerParams(use_tc_tiling_on_sc=False,
                                             needs_layout_passes=False),
    )
    return kern


def kernel(x, table):
    B0, B1 = x.shape
    xt = x.T.astype(jnp.int32)
    out5 = _build(B0, B1)(xt, table)
    out = out5.transpose(2, 4, 0, 1, 3).reshape(B0, B1, D_MODEL)
    return out


# speedup vs baseline: 1.2311x; 1.2311x over previous
import math

import jax
import jax.numpy as jnp
from jax import lax
from jax.experimental import pallas as pl
from jax.experimental.pallas import tpu as pltpu
from jax.experimental.pallas import tpu_sc as plsc

D_MODEL = 64
SCALE = math.sqrt(D_MODEL)

NC = 2
NS = 16
NW = NC * NS
LANES = 16

GATHER = 128
NROWS = 4
NBLK = 2


def _build(B0, B1):
    assert B0 == GATHER * NW and B1 % NROWS == 0
    n_steps = B1
    mesh = plsc.VectorSubcoreMesh(
        core_axis_name="c", subcore_axis_name="s",
        num_cores=NC, num_subcores=NS)

    def body(xt_hbm, table_hbm, out_hbm, idx_all,
             rows0, rows1, rows2, rows3, blk0, blk1,
             sg0, sg1, sg2, sg3, sw0, sw1):
        rows = (rows0, rows1, rows2, rows3)
        blk = (blk0, blk1)
        sem_g = (sg0, sg1, sg2, sg3)
        sem_w = (sw0, sw1)
        wid = lax.axis_index("s") * NC + lax.axis_index("c")
        col0 = wid * GATHER

        pltpu.sync_copy(xt_hbm.at[:, pl.ds(col0, GATHER)], idx_all)

        ilane = lax.iota(jnp.int32, LANES)
        rvecs = [ilane + g * LANES for g in range(GATHER // LANES)]

        def fire_gather(b1, rb):
            pltpu.async_copy(table_hbm.at[idx_all.at[b1]], rows[rb],
                             sem_g[rb])

        def wait_gather(b1, rb):
            pltpu.make_async_copy(table_hbm.at[idx_all.at[b1]], rows[rb],
                                  sem_g[rb]).wait()

        def fire_wb(b1, kb):
            for di in range(D_MODEL // 8):
                pltpu.async_copy(blk[kb].at[pl.ds(di * 8, 8)],
                                 out_hbm.at[b1, di, wid], sem_w[kb])

        def wait_wb(kb):
            for di in range(D_MODEL // 8):
                pltpu.make_async_copy(blk[kb].at[pl.ds(di * 8, 8)],
                                      out_hbm.at[0, di, 0], sem_w[kb]).wait()

        def shuffle(rb, kb):
            src = rows[rb]
            dst = blk[kb]

            @plsc.parallel_loop(0, D_MODEL, unroll=2)
            def _(d):
                cvec = jnp.zeros((LANES,), jnp.int32) + d
                for g in range(GATHER // LANES):
                    v = plsc.load_gather(src, [rvecs[g], cvec])
                    dst[d, pl.ds(g * LANES, LANES)] = v * SCALE

        for rb in range(NROWS):
            fire_gather(rb, rb)

        def quad_body(b1_0, carry):
            for b in range(NROWS):
                b1 = b1_0 + b
                rb = b
                kb = b % NBLK
                wait_gather(b1, rb)

                @pl.when(b1 >= NBLK)
                def _():
                    wait_wb(kb)

                shuffle(rb, kb)
                fire_wb(b1, kb)

                @pl.when(b1 + NROWS < n_steps)
                def _():
                    fire_gather(b1 + NROWS, rb)
            return carry

        lax.fori_loop(0, n_steps // NROWS,
                      lambda i, c: quad_body(i * NROWS, c), 0)
        for kb in range(NBLK):
            wait_wb(kb)

    kern = pl.kernel(
        body,
        out_type=jax.ShapeDtypeStruct((B1, D_MODEL // 8, B0 // GATHER, 8, 128),
                                      jnp.float32),
        mesh=mesh,
        scratch_types=[
            pltpu.VMEM((B1, GATHER), jnp.int32),
            pltpu.VMEM((GATHER, D_MODEL), jnp.float32),
            pltpu.VMEM((GATHER, D_MODEL), jnp.float32),
            pltpu.VMEM((GATHER, D_MODEL), jnp.float32),
            pltpu.VMEM((GATHER, D_MODEL), jnp.float32),
            pltpu.VMEM((D_MODEL, 128), jnp.float32),
            pltpu.VMEM((D_MODEL, 128), jnp.float32),
            pltpu.SemaphoreType.DMA,
            pltpu.SemaphoreType.DMA,
            pltpu.SemaphoreType.DMA,
            pltpu.SemaphoreType.DMA,
            pltpu.SemaphoreType.DMA,
            pltpu.SemaphoreType.DMA,
        ],
        compiler_params=pltpu.CompilerParams(use_tc_tiling_on_sc=False,
                                             needs_layout_passes=False),
    )
    return kern


def kernel(x, table):
    B0, B1 = x.shape
    xt = x.T.astype(jnp.int32)
    out5 = _build(B0, B1)(xt, table)
    out = out5.transpose(2, 4, 0, 1, 3).reshape(B0, B1, D_MODEL)
    return out


# speedup vs baseline: 2.0727x; 1.6835x over previous
import math

import jax
import jax.numpy as jnp
from jax import lax
from jax.experimental import pallas as pl
from jax.experimental.pallas import tpu as pltpu
from jax.experimental.pallas import tpu_sc as plsc

D_MODEL = 64
SCALE = math.sqrt(D_MODEL)

NC = 2
NS = 16
NW = NC * NS
LANES = 16

GATHER = 128
NROWS = 4
NBLK = 2


def _build(B0, B1):
    assert B0 == GATHER * NW and B1 % NROWS == 0
    n_steps = B1
    mesh = plsc.VectorSubcoreMesh(
        core_axis_name="c", subcore_axis_name="s",
        num_cores=NC, num_subcores=NS)

    def body(xt_hbm, table_hbm, out_hbm, idx_all,
             rows0, rows1, rows2, rows3, blk0, blk1,
             sg0, sg1, sg2, sg3, sw0, sw1):
        rows = (rows0, rows1, rows2, rows3)
        blk = (blk0, blk1)
        sem_g = (sg0, sg1, sg2, sg3)
        sem_w = (sw0, sw1)
        wid = lax.axis_index("s") * NC + lax.axis_index("c")
        col0 = wid * GATHER

        pltpu.sync_copy(xt_hbm.at[:, pl.ds(col0, GATHER)], idx_all)

        ilane = lax.iota(jnp.int32, LANES)
        dvecs = [ilane + q * LANES for q in range(D_MODEL // LANES)]

        def fire_gather(b1, rb):
            pltpu.async_copy(table_hbm.at[idx_all.at[b1]], rows[rb],
                             sem_g[rb])

        def wait_gather(b1, rb):
            pltpu.make_async_copy(table_hbm.at[idx_all.at[b1]], rows[rb],
                                  sem_g[rb]).wait()

        def fire_wb(b1, kb):
            for di in range(D_MODEL // 8):
                pltpu.async_copy(
                    blk[kb].at[pl.ds(di * 8, 8), pl.ds(0, GATHER)],
                    out_hbm.at[b1, di, wid], sem_w[kb])

        def wait_wb(kb):
            for di in range(D_MODEL // 8):
                pltpu.make_async_copy(
                    blk[kb].at[pl.ds(di * 8, 8), pl.ds(0, GATHER)],
                    out_hbm.at[0, di, 0], sem_w[kb]).wait()

        def shuffle(rb, kb):
            src = rows[rb]
            dst = blk[kb]

            @plsc.parallel_loop(0, GATHER, unroll=4)
            def _(i):
                ivec = jnp.zeros((LANES,), jnp.int32) + i
                for q in range(D_MODEL // LANES):
                    v = src[i, pl.ds(q * LANES, LANES)]
                    plsc.store_scatter(dst, [dvecs[q], ivec], v * SCALE)

        for rb in range(NROWS):
            fire_gather(rb, rb)

        def quad_body(b1_0, carry):
            for b in range(NROWS):
                b1 = b1_0 + b
                rb = b
                kb = b % NBLK
                wait_gather(b1, rb)

                @pl.when(b1 >= NBLK)
                def _():
                    wait_wb(kb)

                shuffle(rb, kb)
                fire_wb(b1, kb)

                @pl.when(b1 + NROWS < n_steps)
                def _():
                    fire_gather(b1 + NROWS, rb)
            return carry

        lax.fori_loop(0, n_steps // NROWS,
                      lambda i, c: quad_body(i * NROWS, c), 0)
        for kb in range(NBLK):
            wait_wb(kb)

    kern = pl.kernel(
        body,
        out_type=jax.ShapeDtypeStruct((B1, D_MODEL // 8, B0 // GATHER, 8, 128),
                                      jnp.float32),
        mesh=mesh,
        scratch_types=[
            pltpu.VMEM((B1, GATHER), jnp.int32),
            pltpu.VMEM((GATHER, D_MODEL), jnp.float32),
            pltpu.VMEM((GATHER, D_MODEL), jnp.float32),
            pltpu.VMEM((GATHER, D_MODEL), jnp.float32),
            pltpu.VMEM((GATHER, D_MODEL), jnp.float32),
            pltpu.VMEM((D_MODEL, 129), jnp.float32),
            pltpu.VMEM((D_MODEL, 129), jnp.float32),
            pltpu.SemaphoreType.DMA,
            pltpu.SemaphoreType.DMA,
            pltpu.SemaphoreType.DMA,
            pltpu.SemaphoreType.DMA,
            pltpu.SemaphoreType.DMA,
            pltpu.SemaphoreType.DMA,
        ],
        compiler_params=pltpu.CompilerParams(use_tc_tiling_on_sc=False,
                                             needs_layout_passes=False,
                                             skip_device_barrier=True),
    )
    return kern


def kernel(x, table):
    B0, B1 = x.shape
    xt = x.T.astype(jnp.int32)
    out5 = _build(B0, B1)(xt, table)
    out = out5.transpose(2, 4, 0, 1, 3).reshape(B0, B1, D_MODEL)
    return out
